# baseline (speedup 1.0000x reference)
import functools

import jax
import jax.numpy as jnp
from jax.experimental import pallas as pl
from jax.experimental.pallas import tpu as pltpu


def _fused_body(n_experts, eidx_ref, x_ref, w_ref, uin_ref, uout_ref, out_ref):
    h = jax.lax.dot_general(
        x_ref[...], uin_ref[...], (((1,), (0,)), ((), ())),
        preferred_element_type=jnp.float32,
    ).astype(jnp.bfloat16)
    eidx = eidx_ref[...]
    acc = None
    for e in range(n_experts):
        z_e = jax.lax.dot_general(
            w_ref[e], h, (((1,), (1,)), ((), ())),
            preferred_element_type=jnp.float32,
        )
        acc = z_e if acc is None else jnp.where(eidx == e, z_e, acc)
    z_t = acc.astype(jnp.bfloat16)
    out_ref[...] = jax.lax.dot_general(
        z_t, uout_ref[...], (((0,), (1,)), ((), ())),
        preferred_element_type=jnp.float32,
    ).astype(jnp.bfloat16)


@jax.jit
def kernel(x, expert_indices, W_low, U_in, U_out):
    t, d_model = x.shape
    n_experts, u, d = W_low.shape
    tb = 512
    nb = t // tb
    eidx1 = expert_indices.astype(jnp.int32)
    return pl.pallas_call(
        functools.partial(_fused_body, n_experts),
        grid=(nb,),
        in_specs=[
            pl.BlockSpec((tb,), lambda i: (i,)),
            pl.BlockSpec((tb, d_model), lambda i: (i, 0)),
            pl.BlockSpec((n_experts, u, d), lambda i: (0, 0, 0)),
            pl.BlockSpec((d_model, d), lambda i: (0, 0)),
            pl.BlockSpec((d_model, u), lambda i: (0, 0)),
        ],
        out_specs=pl.BlockSpec((tb, d_model), lambda i: (i, 0)),
        out_shape=jax.ShapeDtypeStruct((t, d_model), jnp.bfloat16),
        compiler_params=pltpu.CompilerParams(
            dimension_semantics=("parallel",),
        ),
    )(eidx1, x, W_low, U_in, U_out)


# speedup vs baseline: 1.2786x; 1.2786x over previous
import functools

import jax
import jax.numpy as jnp
from jax.experimental import pallas as pl
from jax.experimental.pallas import tpu as pltpu


def _fused_body(n_experts, eidx_ref, x_ref, w_ref, uin_ref, uout_ref, out_ref):
    h = jax.lax.dot_general(
        x_ref[...], uin_ref[...], (((1,), (0,)), ((), ())),
        preferred_element_type=jnp.float32,
    ).astype(jnp.bfloat16)
    eidx = eidx_ref[0]
    acc = None
    for e in range(n_experts):
        z_e = jax.lax.dot_general(
            w_ref[e], h, (((1,), (1,)), ((), ())),
            preferred_element_type=jnp.float32,
        )
        acc = z_e if acc is None else jnp.where(eidx == e, z_e, acc)
    z_t = acc.astype(jnp.bfloat16)
    out_ref[...] = jax.lax.dot_general(
        z_t, uout_ref[...], (((0,), (1,)), ((), ())),
        preferred_element_type=jnp.float32,
    ).astype(jnp.bfloat16)


@jax.jit
def kernel(x, expert_indices, W_low, U_in, U_out):
    t, d_model = x.shape
    n_experts, u, d = W_low.shape
    tb = 512
    nb = t // tb
    eidx3 = expert_indices.astype(jnp.int32).reshape(nb, 1, tb)
    return pl.pallas_call(
        functools.partial(_fused_body, n_experts),
        grid=(nb,),
        in_specs=[
            pl.BlockSpec((1, 1, tb), lambda i: (i, 0, 0)),
            pl.BlockSpec((tb, d_model), lambda i: (i, 0)),
            pl.BlockSpec((n_experts, u, d), lambda i: (0, 0, 0)),
            pl.BlockSpec((d_model, d), lambda i: (0, 0)),
            pl.BlockSpec((d_model, u), lambda i: (0, 0)),
        ],
        out_specs=pl.BlockSpec((tb, d_model), lambda i: (i, 0)),
        out_shape=jax.ShapeDtypeStruct((t, d_model), jnp.bfloat16),
        compiler_params=pltpu.CompilerParams(
            dimension_semantics=("parallel",),
        ),
    )(eidx3, x, W_low, U_in, U_out)


# speedup vs baseline: 1.2990x; 1.0159x over previous
import functools

import jax
import jax.numpy as jnp
from jax.experimental import pallas as pl
from jax.experimental.pallas import tpu as pltpu


def _fused_body(n_experts, eidx_ref, x_ref, w_ref, uin_ref, uout_ref, out_ref):
    h = jax.lax.dot_general(
        x_ref[...], uin_ref[...], (((1,), (0,)), ((), ())),
        preferred_element_type=jnp.float32,
    ).astype(jnp.bfloat16)
    eidx = eidx_ref[0]
    acc = None
    for e in range(n_experts):
        z_e = jax.lax.dot_general(
            w_ref[e], h, (((1,), (1,)), ((), ())),
            preferred_element_type=jnp.float32,
        )
        acc = z_e if acc is None else jnp.where(eidx == e, z_e, acc)
    z_t = acc.astype(jnp.bfloat16)
    out_ref[...] = jax.lax.dot_general(
        z_t, uout_ref[...], (((0,), (1,)), ((), ())),
        preferred_element_type=jnp.float32,
    ).astype(jnp.bfloat16)


@jax.jit
def kernel(x, expert_indices, W_low, U_in, U_out):
    t, d_model = x.shape
    n_experts, u, d = W_low.shape
    tb = 1024
    nb = t // tb
    eidx3 = expert_indices.astype(jnp.int32).reshape(nb, 1, tb)
    return pl.pallas_call(
        functools.partial(_fused_body, n_experts),
        grid=(nb,),
        in_specs=[
            pl.BlockSpec((1, 1, tb), lambda i: (i, 0, 0)),
            pl.BlockSpec((tb, d_model), lambda i: (i, 0)),
            pl.BlockSpec((n_experts, u, d), lambda i: (0, 0, 0)),
            pl.BlockSpec((d_model, d), lambda i: (0, 0)),
            pl.BlockSpec((d_model, u), lambda i: (0, 0)),
        ],
        out_specs=pl.BlockSpec((tb, d_model), lambda i: (i, 0)),
        out_shape=jax.ShapeDtypeStruct((t, d_model), jnp.bfloat16),
        compiler_params=pltpu.CompilerParams(
            dimension_semantics=("parallel",),
        ),
    )(eidx3, x, W_low, U_in, U_out)
